# Initial kernel scaffold; baseline (speedup 1.0000x reference)
import jax
import jax.numpy as jnp
from jax.experimental import pallas as pl


def kernel(token_ids, embedding_mat):
    raise NotImplementedError("write your pallas kernel here")


# speedup vs baseline: 2.9763x; 2.9763x over previous
import functools

import jax
import jax.numpy as jnp
from jax import lax
from jax.experimental import pallas as pl
from jax.experimental.pallas import tpu as pltpu
from jax.experimental.pallas import tpu_sc as plsc

NUM_EMBEDDINGS = 100000
EMBEDDING_DIM = 128
BATCH = 4096
HIST_LEN = 50

NC = 2
NS = 16
NW = NC * NS

TOTAL = BATCH * HIST_LEN
CHUNK = 128
CHUNKS_PER_W = TOTAL // (NW * CHUNK)


def _make_kernel():
    mesh = plsc.VectorSubcoreMesh(
        core_axis_name="c", subcore_axis_name="s",
        num_cores=NC, num_subcores=NS)

    @functools.partial(
        pl.kernel,
        out_type=jax.ShapeDtypeStruct((TOTAL, EMBEDDING_DIM), jnp.float32),
        mesh=mesh,
        scratch_types=[
            pltpu.VMEM((1, CHUNKS_PER_W, CHUNK), jnp.int32),
            pltpu.VMEM((CHUNK, EMBEDDING_DIM), jnp.float32),
            pltpu.SemaphoreType.DMA,
        ],
    )
    def gather_kernel(idx_hbm, table_hbm, out_hbm, idx_v, rows_v, sem):
        wid = lax.axis_index("s") * NC + lax.axis_index("c")
        pltpu.sync_copy(idx_hbm.at[pl.ds(wid, 1)], idx_v)

        def step(j, carry):
            g = wid * CHUNKS_PER_W + j
            pltpu.async_copy(table_hbm.at[idx_v.at[0, j]], rows_v, sem).wait()
            pltpu.sync_copy(rows_v, out_hbm.at[pl.ds(g * CHUNK, CHUNK)])
            return carry

        lax.fori_loop(0, CHUNKS_PER_W, step, 0)

    return gather_kernel


_gather = _make_kernel()


def kernel(token_ids, embedding_mat):
    idx = jnp.reshape(token_ids.astype(jnp.int32), (NW, CHUNKS_PER_W, CHUNK))
    out = _gather(idx, embedding_mat)
    return jnp.reshape(out, (BATCH, HIST_LEN, EMBEDDING_DIM))


# speedup vs baseline: 3.3440x; 1.1235x over previous
import functools

import jax
import jax.numpy as jnp
from jax import lax
from jax.experimental import pallas as pl
from jax.experimental.pallas import tpu as pltpu
from jax.experimental.pallas import tpu_sc as plsc

NUM_EMBEDDINGS = 100000
EMBEDDING_DIM = 128
BATCH = 4096
HIST_LEN = 50

NC = 2
NS = 16
NW = NC * NS

TOTAL = BATCH * HIST_LEN
CHUNK = 128
CHUNKS_PER_W = TOTAL // (NW * CHUNK)
NBUF = 5


def _make_kernel():
    mesh = plsc.VectorSubcoreMesh(
        core_axis_name="c", subcore_axis_name="s",
        num_cores=NC, num_subcores=NS)

    @functools.partial(
        pl.kernel,
        out_type=jax.ShapeDtypeStruct((TOTAL, EMBEDDING_DIM), jnp.float32),
        mesh=mesh,
        scratch_types=(
            [pltpu.VMEM((1, CHUNKS_PER_W, CHUNK), jnp.int32)]
            + [pltpu.VMEM((CHUNK, EMBEDDING_DIM), jnp.float32)
               for _ in range(NBUF)]
            + [pltpu.SemaphoreType.DMA for _ in range(2 * NBUF)]
        ),
    )
    def gather_kernel(idx_hbm, table_hbm, out_hbm, idx_v, *scratch):
        bufs = scratch[:NBUF]
        gsem = scratch[NBUF:2 * NBUF]
        psem = scratch[2 * NBUF:]
        wid = lax.axis_index("s") * NC + lax.axis_index("c")
        pltpu.sync_copy(idx_hbm.at[pl.ds(wid, 1)], idx_v)

        for b in range(NBUF):
            pltpu.async_copy(table_hbm.at[idx_v.at[0, b]], bufs[b], gsem[b])

        def outer(t, carry):
            for b in range(NBUF):
                j = t * NBUF + b
                g = wid * CHUNKS_PER_W + j
                pltpu.make_async_copy(
                    table_hbm.at[idx_v.at[0, 0]], bufs[b], gsem[b]).wait()
                pltpu.async_copy(
                    bufs[b], out_hbm.at[pl.ds(g * CHUNK, CHUNK)], psem[b])
                jn = j + NBUF

                @pl.when(jn < CHUNKS_PER_W)
                def _():
                    pltpu.make_async_copy(
                        bufs[b], out_hbm.at[pl.ds(0, CHUNK)], psem[b]).wait()
                    pltpu.async_copy(
                        table_hbm.at[idx_v.at[0, jn]], bufs[b], gsem[b])
            return carry

        lax.fori_loop(0, CHUNKS_PER_W // NBUF, outer, 0)
        for b in range(NBUF):
            pltpu.make_async_copy(
                bufs[b], out_hbm.at[pl.ds(0, CHUNK)], psem[b]).wait()

    return gather_kernel


_gather = _make_kernel()


def kernel(token_ids, embedding_mat):
    idx = jnp.reshape(token_ids.astype(jnp.int32), (NW, CHUNKS_PER_W, CHUNK))
    out = _gather(idx, embedding_mat)
    return jnp.reshape(out, (BATCH, HIST_LEN, EMBEDDING_DIM))


# speedup vs baseline: 5.9671x; 1.7844x over previous
import functools

import jax
import jax.numpy as jnp
from jax import lax
from jax.experimental import pallas as pl
from jax.experimental.pallas import tpu as pltpu
from jax.experimental.pallas import tpu_sc as plsc

NUM_EMBEDDINGS = 100000
EMBEDDING_DIM = 128
BATCH = 4096
HIST_LEN = 50

NC = 2
NS = 16
NW = NC * NS

B_PER_W = BATCH // NW
GB = 8
GROUPS = B_PER_W // GB
NBUF = 2


def _make_kernel():
    mesh = plsc.VectorSubcoreMesh(
        core_axis_name="c", subcore_axis_name="s",
        num_cores=NC, num_subcores=NS)

    @functools.partial(
        pl.kernel,
        out_type=jax.ShapeDtypeStruct((BATCH, HIST_LEN, EMBEDDING_DIM),
                                      jnp.float32),
        mesh=mesh,
        scratch_types=(
            [pltpu.VMEM((1, B_PER_W, HIST_LEN), jnp.int32)]
            + [pltpu.VMEM((GB, HIST_LEN, EMBEDDING_DIM), jnp.float32)
               for _ in range(NBUF)]
            + [pltpu.SemaphoreType.DMA for _ in range(2 * NBUF)]
        ),
    )
    def gather_kernel(idx_hbm, table_hbm, out_hbm, idx_v, *scratch):
        bufs = scratch[:NBUF]
        gsem = scratch[NBUF:2 * NBUF]
        psem = scratch[2 * NBUF:]
        wid = lax.axis_index("s") * NC + lax.axis_index("c")
        pltpu.sync_copy(idx_hbm.at[pl.ds(wid, 1)], idx_v)

        def gather_group(grp, b):
            for i in range(GB):
                pltpu.async_copy(
                    table_hbm.at[idx_v.at[0, grp * GB + i]],
                    bufs[b].at[i], gsem[b])

        def wait_group(b):
            for i in range(GB):
                pltpu.make_async_copy(
                    table_hbm.at[idx_v.at[0, 0]],
                    bufs[b].at[i], gsem[b]).wait()

        for b in range(NBUF):
            gather_group(b, b)

        def outer(t, carry):
            for b in range(NBUF):
                grp = t * NBUF + b
                base = wid * B_PER_W + grp * GB
                wait_group(b)
                pltpu.async_copy(
                    bufs[b], out_hbm.at[pl.ds(base, GB)], psem[b])
                gn = grp + NBUF

                @pl.when(gn < GROUPS)
                def _():
                    pltpu.make_async_copy(
                        bufs[b], out_hbm.at[pl.ds(0, GB)], psem[b]).wait()
                    gather_group(gn, b)
            return carry

        lax.fori_loop(0, GROUPS // NBUF, outer, 0)
        for b in range(NBUF):
            pltpu.make_async_copy(
                bufs[b], out_hbm.at[pl.ds(0, GB)], psem[b]).wait()

    return gather_kernel


_gather = _make_kernel()


def kernel(token_ids, embedding_mat):
    idx = jnp.reshape(token_ids.astype(jnp.int32), (NW, B_PER_W, HIST_LEN))
    return _gather(idx, embedding_mat)
